# baseline (speedup 1.0000x reference)
import functools

import jax
import jax.numpy as jnp
import numpy as np
from jax import lax
from jax.experimental import pallas as pl
from jax.experimental.pallas import tpu as pltpu
from jax.experimental.pallas import tpu_sc as plsc

NC = 2
NS = 16
NW = NC * NS
CHUNK = 40
L = 16
D = 128
DROWS = 128
DLROWS = 80


def _qkv_body(x_ref, wq_ref, wk_ref, wv_ref, bq_ref, bk_ref, bv_ref,
              q_ref, k_ref, v_ref):
    x = x_ref[...]
    dn = (((1,), (1,)), ((), ()))
    q_ref[...] = lax.dot_general(x, wq_ref[...], dn,
                                 preferred_element_type=jnp.float32,
                                 precision=lax.Precision.HIGHEST) + bq_ref[...]
    k_ref[...] = lax.dot_general(x, wk_ref[...], dn,
                                 preferred_element_type=jnp.float32,
                                 precision=lax.Precision.HIGHEST) + bk_ref[...]
    v_ref[...] = lax.dot_general(x, wv_ref[...], dn,
                                 preferred_element_type=jnp.float32,
                                 precision=lax.Precision.HIGHEST) + bv_ref[...]


def _out_body(h0_ref, h1_ref, d0_ref, d1_ref, wo_ref, bo_ref, out_ref):
    h = h0_ref[...] + h1_ref[...]
    den = d0_ref[...] + d1_ref[...] + jnp.float32(1e-16)
    hn = h / den
    dn = (((1,), (1,)), ((), ()))
    out_ref[...] = lax.dot_general(hn, wo_ref[...], dn,
                                   preferred_element_type=jnp.float32,
                                   precision=lax.Precision.HIGHEST) + bo_ref[...]


def _lane_take(vec, idx):
    dnums = lax.GatherDimensionNumbers(
        offset_dims=(), collapsed_slice_dims=(0,), start_index_map=(0,))
    return lax.gather(vec, idx[:, None], dnums, (1,),
                      mode=lax.GatherScatterMode.PROMISE_IN_BOUNDS)


def _edge_sc_call(q, k, v, src, dst):
    n = q.shape[0]
    e = src.shape[0]
    epw = e // NW
    nchunk = epw // CHUNK
    npad = ((n + NS * 8 - 1) // (NS * 8)) * (NS * 8)
    nrows = npad + DROWS
    rows_per_tile = nrows // NS
    mesh = plsc.VectorSubcoreMesh(core_axis_name="c", subcore_axis_name="s",
                                  num_cores=NC, num_subcores=NS)

    body = functools.partial(_edge_sc_body_impl, npad, epw, nchunk,
                             rows_per_tile)

    fn = pl.kernel(
        body,
        out_type=jax.ShapeDtypeStruct((NC, nrows, D), jnp.float32),
        mesh=mesh,
        scratch_types=[
            pltpu.VMEM((2, CHUNK), jnp.int32),
            pltpu.VMEM((2, CHUNK + L), jnp.int32),
            pltpu.VMEM((2, CHUNK), jnp.int32),
            pltpu.VMEM((6 * CHUNK, D), jnp.float32),
            pltpu.VMEM((CHUNK, D), jnp.float32),
            pltpu.VMEM((DLROWS, D), jnp.float32),
            pltpu.VMEM((DLROWS,), jnp.int32),
            pltpu.VMEM_SHARED((nrows, D), jnp.float32),
            pltpu.SemaphoreType.DMA,
            pltpu.SemaphoreType.DMA,
        ],
    )
    return fn(q, k, v, src, dst), npad


def _edge_sc_body_impl(npad, epw, nchunk, rows_per_tile,
                       q_hbm, k_hbm, v_hbm, src_hbm, dst_hbm, hd_out,
                       sidx, didx, dscat, gbuf, wvbuf, dloc, dconst, hacc,
                       gsem, isem):
    c = lax.axis_index("c")
    s = lax.axis_index("s")
    wid = s * NC + c
    zero16 = jnp.zeros((L,), jnp.float32)
    lane = lax.broadcasted_iota(jnp.int32, (L,), 0)
    nz = 128
    inv = jnp.float32(1.0 / np.sqrt(D))
    ebase = wid * epw

    def dzero(i, carry):
        for j in range(D // L):
            dloc[i, pl.ds(j * L, L)] = zero16
        return carry
    lax.fori_loop(0, DLROWS, dzero, 0)
    for g in range(DLROWS // L):
        dconst[pl.ds(g * L, L)] = lane + (npad + g * L)

    def gzero(i, carry):
        for j in range(D // L):
            gbuf[i, pl.ds(j * L, L)] = zero16
        return carry
    lax.fori_loop(0, nz, gzero, 0)

    def zcopy(r, carry):
        base = s * rows_per_tile + r * nz
        pltpu.sync_copy(gbuf.at[pl.ds(0, nz), :],
                        hacc.at[pl.ds(base, nz), :])
        return carry
    lax.fori_loop(0, rows_per_tile // nz, zcopy, 0)
    plsc.subcore_barrier()

    def issue_idx(i, sync):
        off = ebase + i * CHUNK
        b = lax.rem(i, 2)
        if sync:
            pltpu.sync_copy(src_hbm.at[pl.ds(off, CHUNK)], sidx.at[b])
            pltpu.sync_copy(dst_hbm.at[pl.ds(off, CHUNK)],
                            didx.at[b, pl.ds(0, CHUNK)])
            pltpu.sync_copy(dst_hbm.at[pl.ds(off, CHUNK)], dscat.at[b])
        else:
            pltpu.async_copy(src_hbm.at[pl.ds(off, CHUNK)], sidx.at[b],
                             isem)
            pltpu.async_copy(dst_hbm.at[pl.ds(off, CHUNK)],
                             didx.at[b, pl.ds(0, CHUNK)], isem)
            pltpu.async_copy(dst_hbm.at[pl.ds(off, CHUNK)], dscat.at[b],
                             isem)

    def wait_idx(i):
        off = ebase + i * CHUNK
        b = lax.rem(i, 2)
        pltpu.make_async_copy(src_hbm.at[pl.ds(off, CHUNK)], sidx.at[b],
                              isem).wait()
        pltpu.make_async_copy(dst_hbm.at[pl.ds(off, CHUNK)],
                              didx.at[b, pl.ds(0, CHUNK)], isem).wait()
        pltpu.make_async_copy(dst_hbm.at[pl.ds(off, CHUNK)], dscat.at[b],
                              isem).wait()

    def issue_gathers(i):
        b = lax.rem(i, 2)
        gb = b * (3 * CHUNK)
        pltpu.async_copy(q_hbm.at[dscat.at[b]],
                         gbuf.at[pl.ds(gb, CHUNK), :], gsem)
        pltpu.async_copy(k_hbm.at[sidx.at[b]],
                         gbuf.at[pl.ds(gb + CHUNK, CHUNK), :], gsem)
        pltpu.async_copy(v_hbm.at[sidx.at[b]],
                         gbuf.at[pl.ds(gb + 2 * CHUNK, CHUNK), :], gsem)

    def wait_gathers(i):
        b = lax.rem(i, 2)
        gb = b * (3 * CHUNK)
        pltpu.make_async_copy(q_hbm.at[dscat.at[b]],
                              gbuf.at[pl.ds(gb, CHUNK), :], gsem).wait()
        pltpu.make_async_copy(k_hbm.at[sidx.at[b]],
                              gbuf.at[pl.ds(gb + CHUNK, CHUNK), :],
                              gsem).wait()
        pltpu.make_async_copy(v_hbm.at[sidx.at[b]],
                              gbuf.at[pl.ds(gb + 2 * CHUNK, CHUNK), :],
                              gsem).wait()

    issue_idx(0, True)
    issue_gathers(0)
    issue_idx(1, False)

    def chunk_body(i, carry):
        b = lax.rem(i, 2)
        gb = b * (3 * CHUNK)

        @pl.when(i < nchunk - 1)
        def _():
            wait_idx(i + 1)
            issue_gathers(i + 1)
        wait_gathers(i)

        def edge_body(e4, inner):
          for u in range(4):
            e = e4 * 4 + u
            acc_a = (gbuf[gb + e, pl.ds(0, L)] *
                     gbuf[gb + CHUNK + e, pl.ds(0, L)])
            acc_b = (gbuf[gb + e, pl.ds(4 * L, L)] *
                     gbuf[gb + CHUNK + e, pl.ds(4 * L, L)])
            for j in range(1, 4):
                acc_a = acc_a + (gbuf[gb + e, pl.ds(j * L, L)] *
                                 gbuf[gb + CHUNK + e, pl.ds(j * L, L)])
                acc_b = acc_b + (gbuf[gb + e, pl.ds((4 + j) * L, L)] *
                                 gbuf[gb + CHUNK + e, pl.ds((4 + j) * L, L)])
            acc = acc_a + acc_b
            for sh in (8, 4, 2, 1):
                acc = acc + _lane_take(acc, lane ^ sh)
            w = jnp.exp(acc * inv)
            for j in range(D // L):
                wvbuf[e, pl.ds(j * L, L)] = (
                    gbuf[gb + 2 * CHUNK + e, pl.ds(j * L, L)] * w)
            d = didx[b, pl.ds(e, L)][0]
            row = d >> 7
            col = d & 127
            vsel = (col >> 4) * L
            onehot = jnp.where(lane == (col & (L - 1)), jnp.float32(1.0),
                               jnp.float32(0.0))
            cur = dloc[row, pl.ds(vsel, L)]
            dloc[row, pl.ds(vsel, L)] = cur + w * onehot
          return inner
        lax.fori_loop(0, CHUNK // 4, edge_body, 0)

        pltpu.sync_copy(wvbuf, hacc.at[dscat.at[b]], add=True)

        @pl.when(i < nchunk - 2)
        def _():
            issue_idx(i + 2, False)
        return carry
    lax.fori_loop(0, nchunk, chunk_body, 0)

    pltpu.sync_copy(dloc, hacc.at[dconst], add=True)
    plsc.subcore_barrier()

    def wcopy(r, carry):
        base = s * rows_per_tile + r * nz
        pltpu.sync_copy(hacc.at[pl.ds(base, nz), :],
                        gbuf.at[pl.ds(0, nz), :])
        pltpu.sync_copy(gbuf.at[pl.ds(0, nz), :],
                        hd_out.at[c, pl.ds(base, nz), :])
        return carry
    lax.fori_loop(0, rows_per_tile // nz, wcopy, 0)


def kernel(x, edge_index, Wq, bq, Wk, bk, Wv, bv, Wo, bo):
    n = x.shape[0]
    rblk = 2000
    grid = n // rblk

    wspec = pl.BlockSpec((D, D), lambda i: (0, 0))
    bspec = pl.BlockSpec((1, D), lambda i: (0, 0))
    rowspec = pl.BlockSpec((rblk, D), lambda i: (i, 0))

    q, k, v = pl.pallas_call(
        _qkv_body,
        grid=(grid,),
        in_specs=[rowspec, wspec, wspec, wspec, bspec, bspec, bspec],
        out_specs=[rowspec, rowspec, rowspec],
        out_shape=[jax.ShapeDtypeStruct((n, D), jnp.float32)] * 3,
    )(x, Wq, Wk, Wv, bq.reshape(1, D), bk.reshape(1, D), bv.reshape(1, D))

    src = edge_index[0]
    dst = edge_index[1]
    hd, npad = _edge_sc_call(q, k, v, src, dst)

    h0 = hd[0, :npad, :]
    h1 = hd[1, :npad, :]
    d0 = hd[0, npad:, :].reshape(DROWS * D)[:npad].reshape(npad, 1)
    d1 = hd[1, npad:, :].reshape(DROWS * D)[:npad].reshape(npad, 1)

    pblk = npad // 8
    prowspec = pl.BlockSpec((pblk, D), lambda i: (i, 0))
    pdspec = pl.BlockSpec((pblk, 1), lambda i: (i, 0))
    out = pl.pallas_call(
        _out_body,
        grid=(8,),
        in_specs=[prowspec, prowspec, pdspec, pdspec, wspec, bspec],
        out_specs=prowspec,
        out_shape=jax.ShapeDtypeStruct((npad, D), jnp.float32),
    )(h0, h1, d0, d1, Wo, bo.reshape(1, D))
    return out[:n]


# speedup vs baseline: 1.3893x; 1.3893x over previous
import functools

import jax
import jax.numpy as jnp
import numpy as np
from jax import lax
from jax.experimental import pallas as pl
from jax.experimental.pallas import tpu as pltpu
from jax.experimental.pallas import tpu_sc as plsc

NC = 2
NS = 16
NW = NC * NS
CHUNK = 40
L = 16
D = 128
DROWS = 128
DLROWS = 80


def _qkv_body(x_ref, wq_ref, wk_ref, wv_ref, bq_ref, bk_ref, bv_ref,
              q_ref, k_ref, v_ref):
    x = x_ref[...]
    dn = (((1,), (1,)), ((), ()))
    q_ref[...] = lax.dot_general(x, wq_ref[...], dn,
                                 preferred_element_type=jnp.float32,
                                 precision=lax.Precision.HIGHEST) + bq_ref[...]
    k_ref[...] = lax.dot_general(x, wk_ref[...], dn,
                                 preferred_element_type=jnp.float32,
                                 precision=lax.Precision.HIGHEST) + bk_ref[...]
    v_ref[...] = lax.dot_general(x, wv_ref[...], dn,
                                 preferred_element_type=jnp.float32,
                                 precision=lax.Precision.HIGHEST) + bv_ref[...]


def _out_body(h0_ref, h1_ref, d0_ref, d1_ref, wo_ref, bo_ref, out_ref):
    h = h0_ref[...] + h1_ref[...]
    den = d0_ref[...] + d1_ref[...] + jnp.float32(1e-16)
    hn = h / den
    dn = (((1,), (1,)), ((), ()))
    out_ref[...] = lax.dot_general(hn, wo_ref[...], dn,
                                   preferred_element_type=jnp.float32,
                                   precision=lax.Precision.HIGHEST) + bo_ref[...]


def _lane_take(vec, idx):
    dnums = lax.GatherDimensionNumbers(
        offset_dims=(), collapsed_slice_dims=(0,), start_index_map=(0,))
    return lax.gather(vec, idx[:, None], dnums, (1,),
                      mode=lax.GatherScatterMode.PROMISE_IN_BOUNDS)


def _edge_sc_call(q, k, v, src, dst):
    n = q.shape[0]
    e = src.shape[0]
    epw = e // NW
    nchunk = epw // CHUNK
    npad = ((n + NS * 8 - 1) // (NS * 8)) * (NS * 8)
    nrows = npad + DROWS
    rows_per_tile = nrows // NS
    mesh = plsc.VectorSubcoreMesh(core_axis_name="c", subcore_axis_name="s",
                                  num_cores=NC, num_subcores=NS)

    body = functools.partial(_edge_sc_body_impl, npad, epw, nchunk,
                             rows_per_tile)

    fn = pl.kernel(
        body,
        out_type=jax.ShapeDtypeStruct((NC, nrows, D), jnp.float32),
        mesh=mesh,
        scratch_types=[
            pltpu.VMEM((2, CHUNK), jnp.int32),
            pltpu.VMEM((2, CHUNK + L), jnp.int32),
            pltpu.VMEM((2, CHUNK), jnp.int32),
            pltpu.VMEM((6 * CHUNK, D), jnp.float32),
            pltpu.VMEM((CHUNK, D), jnp.float32),
            pltpu.VMEM((DLROWS, D), jnp.float32),
            pltpu.VMEM((DLROWS,), jnp.int32),
            pltpu.VMEM_SHARED((nrows, D), jnp.float32),
            pltpu.SemaphoreType.DMA,
            pltpu.SemaphoreType.DMA,
        ],
    )
    return fn(q, k, v, src, dst), npad


def _edge_sc_body_impl(npad, epw, nchunk, rows_per_tile,
                       q_hbm, k_hbm, v_hbm, src_hbm, dst_hbm, hd_out,
                       sidx, didx, dscat, gbuf, wvbuf, dloc, dconst, hacc,
                       gsem, isem):
    c = lax.axis_index("c")
    s = lax.axis_index("s")
    wid = s * NC + c
    zero16 = jnp.zeros((L,), jnp.float32)
    lane = lax.broadcasted_iota(jnp.int32, (L,), 0)
    nz = 128
    inv = jnp.float32(1.0 / np.sqrt(D))
    ebase = wid * epw

    def dzero(i, carry):
        for j in range(D // L):
            dloc[i, pl.ds(j * L, L)] = zero16
        return carry
    lax.fori_loop(0, DLROWS, dzero, 0)
    for g in range(DLROWS // L):
        dconst[pl.ds(g * L, L)] = lane + (npad + g * L)

    def gzero(i, carry):
        for j in range(D // L):
            gbuf[i, pl.ds(j * L, L)] = zero16
        return carry
    lax.fori_loop(0, nz, gzero, 0)

    def zcopy(r, carry):
        base = s * rows_per_tile + r * nz
        pltpu.sync_copy(gbuf.at[pl.ds(0, nz), :],
                        hacc.at[pl.ds(base, nz), :])
        return carry
    lax.fori_loop(0, rows_per_tile // nz, zcopy, 0)
    plsc.subcore_barrier()

    def issue_idx(i, sync):
        off = ebase + i * CHUNK
        b = lax.rem(i, 2)
        if sync:
            pltpu.sync_copy(src_hbm.at[pl.ds(off, CHUNK)], sidx.at[b])
            pltpu.sync_copy(dst_hbm.at[pl.ds(off, CHUNK)],
                            didx.at[b, pl.ds(0, CHUNK)])
            pltpu.sync_copy(dst_hbm.at[pl.ds(off, CHUNK)], dscat.at[b])
        else:
            pltpu.async_copy(src_hbm.at[pl.ds(off, CHUNK)], sidx.at[b],
                             isem)
            pltpu.async_copy(dst_hbm.at[pl.ds(off, CHUNK)],
                             didx.at[b, pl.ds(0, CHUNK)], isem)
            pltpu.async_copy(dst_hbm.at[pl.ds(off, CHUNK)], dscat.at[b],
                             isem)

    def wait_idx(i):
        off = ebase + i * CHUNK
        b = lax.rem(i, 2)
        pltpu.make_async_copy(src_hbm.at[pl.ds(off, CHUNK)], sidx.at[b],
                              isem).wait()
        pltpu.make_async_copy(dst_hbm.at[pl.ds(off, CHUNK)],
                              didx.at[b, pl.ds(0, CHUNK)], isem).wait()
        pltpu.make_async_copy(dst_hbm.at[pl.ds(off, CHUNK)], dscat.at[b],
                              isem).wait()

    def issue_gathers(i):
        b = lax.rem(i, 2)
        gb = b * (3 * CHUNK)
        pltpu.async_copy(q_hbm.at[dscat.at[b]],
                         gbuf.at[pl.ds(gb, CHUNK), :], gsem)
        pltpu.async_copy(k_hbm.at[sidx.at[b]],
                         gbuf.at[pl.ds(gb + CHUNK, CHUNK), :], gsem)
        pltpu.async_copy(v_hbm.at[sidx.at[b]],
                         gbuf.at[pl.ds(gb + 2 * CHUNK, CHUNK), :], gsem)

    def wait_gathers(i):
        b = lax.rem(i, 2)
        gb = b * (3 * CHUNK)
        pltpu.make_async_copy(q_hbm.at[dscat.at[b]],
                              gbuf.at[pl.ds(gb, CHUNK), :], gsem).wait()
        pltpu.make_async_copy(k_hbm.at[sidx.at[b]],
                              gbuf.at[pl.ds(gb + CHUNK, CHUNK), :],
                              gsem).wait()
        pltpu.make_async_copy(v_hbm.at[sidx.at[b]],
                              gbuf.at[pl.ds(gb + 2 * CHUNK, CHUNK), :],
                              gsem).wait()

    issue_idx(0, True)
    issue_gathers(0)
    issue_idx(1, False)

    def chunk_body(i, carry):
        b = lax.rem(i, 2)
        gb = b * (3 * CHUNK)

        @pl.when(i < nchunk - 1)
        def _():
            wait_idx(i + 1)
            issue_gathers(i + 1)
        wait_gathers(i)

        def edge_body(e4, inner):
            es = [4 * e4 + u for u in range(4)]
            dv = didx[b, pl.ds(4 * e4, L)]
            acc = [None] * 4
            for j in range(D // L):
                for u in range(4):
                    prod = (gbuf[gb + es[u], pl.ds(j * L, L)] *
                            gbuf[gb + CHUNK + es[u], pl.ds(j * L, L)])
                    acc[u] = prod if j == 0 else acc[u] + prod
            for sh in (8, 4, 2, 1):
                for u in range(4):
                    acc[u] = acc[u] + _lane_take(acc[u], lane ^ sh)
            w = [jnp.exp(acc[u] * inv) for u in range(4)]
            for j in range(D // L):
                for u in range(4):
                    wvbuf[es[u], pl.ds(j * L, L)] = (
                        gbuf[gb + 2 * CHUNK + es[u], pl.ds(j * L, L)] * w[u])
            for u in range(4):
                d = dv[u]
                row = d >> 7
                col = d & 127
                vsel = (col >> 4) * L
                onehot = jnp.where(lane == (col & (L - 1)),
                                   jnp.float32(1.0), jnp.float32(0.0))
                cur = dloc[row, pl.ds(vsel, L)]
                dloc[row, pl.ds(vsel, L)] = cur + w[u] * onehot
            return inner
        lax.fori_loop(0, CHUNK // 4, edge_body, 0)

        pltpu.sync_copy(wvbuf, hacc.at[dscat.at[b]], add=True)

        @pl.when(i < nchunk - 2)
        def _():
            issue_idx(i + 2, False)
        return carry
    lax.fori_loop(0, nchunk, chunk_body, 0)

    pltpu.sync_copy(dloc, hacc.at[dconst], add=True)
    plsc.subcore_barrier()

    def wcopy(r, carry):
        base = s * rows_per_tile + r * nz
        pltpu.sync_copy(hacc.at[pl.ds(base, nz), :],
                        gbuf.at[pl.ds(0, nz), :])
        pltpu.sync_copy(gbuf.at[pl.ds(0, nz), :],
                        hd_out.at[c, pl.ds(base, nz), :])
        return carry
    lax.fori_loop(0, rows_per_tile // nz, wcopy, 0)


def kernel(x, edge_index, Wq, bq, Wk, bk, Wv, bv, Wo, bo):
    n = x.shape[0]
    rblk = 2000
    grid = n // rblk

    wspec = pl.BlockSpec((D, D), lambda i: (0, 0))
    bspec = pl.BlockSpec((1, D), lambda i: (0, 0))
    rowspec = pl.BlockSpec((rblk, D), lambda i: (i, 0))

    q, k, v = pl.pallas_call(
        _qkv_body,
        grid=(grid,),
        in_specs=[rowspec, wspec, wspec, wspec, bspec, bspec, bspec],
        out_specs=[rowspec, rowspec, rowspec],
        out_shape=[jax.ShapeDtypeStruct((n, D), jnp.float32)] * 3,
    )(x, Wq, Wk, Wv, bq.reshape(1, D), bk.reshape(1, D), bv.reshape(1, D))

    src = edge_index[0]
    dst = edge_index[1]
    hd, npad = _edge_sc_call(q, k, v, src, dst)

    h0 = hd[0, :npad, :]
    h1 = hd[1, :npad, :]
    d0 = hd[0, npad:, :].reshape(DROWS * D)[:npad].reshape(npad, 1)
    d1 = hd[1, npad:, :].reshape(DROWS * D)[:npad].reshape(npad, 1)

    pblk = npad // 8
    prowspec = pl.BlockSpec((pblk, D), lambda i: (i, 0))
    pdspec = pl.BlockSpec((pblk, 1), lambda i: (i, 0))
    out = pl.pallas_call(
        _out_body,
        grid=(8,),
        in_specs=[prowspec, prowspec, pdspec, pdspec, wspec, bspec],
        out_specs=prowspec,
        out_shape=jax.ShapeDtypeStruct((npad, D), jnp.float32),
    )(h0, h1, d0, d1, Wo, bo.reshape(1, D))
    return out[:n]
